# baseline (speedup 1.0000x reference)
import jax
import jax.numpy as jnp
from jax.experimental import pallas as pl
from jax.experimental.pallas import tpu as pltpu


def _vq_body(z_ref, emb_ref, emb2_ref, zq_ref, loss_ref, closs_ref, qloss_ref,
             idx_ref):
    E = emb_ref[...]
    E2 = emb2_ref[...]
    e_sq = jnp.sum(E * E, axis=1, keepdims=True)
    for bb in range(z_ref.shape[0]):
        _vq_batch(z_ref, emb_ref, emb2_ref, zq_ref, loss_ref, closs_ref,
                  qloss_ref, idx_ref, E, E2, e_sq, bb)


def _vq_batch(z_ref, emb_ref, emb2_ref, zq_ref, loss_ref, closs_ref, qloss_ref,
              idx_ref, E, E2, e_sq, bb):
    Z = z_ref[bb]
    hw = Z.shape[1]
    n = E.shape[0]
    z_sq = jnp.sum(Z * Z, axis=0, keepdims=True)
    mm2 = jax.lax.dot_general(
        E2, Z, (((1,), (0,)), ((), ())),
        preferred_element_type=jnp.float32)

    nchains = 4
    ngroups = n // 8
    sub = jax.lax.broadcasted_iota(jnp.int32, (8, hw), 0)
    chains = []
    for k in range(nchains):
        run = None
        rid = None
        for r in range(k, ngroups, nchains):
            dr = (z_sq + e_sq[r * 8:(r + 1) * 8]) - mm2[r * 8:(r + 1) * 8]
            if run is None:
                run, rid = dr, jnp.full((8, hw), r, jnp.int32)
            else:
                lt = dr < run
                rid = jnp.where(lt, r, rid)
                run = jnp.minimum(run, dr)
        chains.append((run, rid * 8 + sub))

    def merge(a, b):
        take_b = (b[0] < a[0]) | ((b[0] == a[0]) & (b[1] < a[1]))
        return (jnp.where(take_b, b[0], a[0]),
                jnp.where(take_b, b[1], a[1]))

    run, code = chains[0]
    for ch in chains[1:]:
        run, code = merge((run, code), ch)

    vmin = jnp.min(run, axis=0, keepdims=True)
    idx = jnp.min(jnp.where(run == vmin, code, jnp.int32(n)),
                  axis=0, keepdims=True)

    iota = jax.lax.broadcasted_iota(jnp.int32, (n, hw), 0)
    onehot = (iota == idx).astype(jnp.float32)
    zq_t = jax.lax.dot_general(
        E, onehot, (((0,), (0,)), ((), ())),
        preferred_element_type=jnp.float32)
    zq_ref[bb] = zq_t
    diff = zq_t - Z
    sq_t = diff * diff
    sq = sq_t.T
    loss_ref[bb] = 1.25 * sq
    closs_ref[bb] = 0.25 * sq
    qloss_ref[bb] = sq
    idx_ref[bb] = idx


def kernel(z, embedding):
    z = z.astype(jnp.float32)
    b, c, h, w = z.shape
    hw = h * w
    n = embedding.shape[0]
    z3 = z.reshape(b, c, hw)
    emb2 = embedding + embedding
    bb = 4

    out_shapes = (
        jax.ShapeDtypeStruct((b, c, hw), jnp.float32),
        jax.ShapeDtypeStruct((b, hw, c), jnp.float32),
        jax.ShapeDtypeStruct((b, hw, c), jnp.float32),
        jax.ShapeDtypeStruct((b, hw, c), jnp.float32),
        jax.ShapeDtypeStruct((b, 1, hw), jnp.int32),
    )
    zq, loss, closs, qloss, idx = pl.pallas_call(
        _vq_body,
        grid=(b // bb,),
        in_specs=[
            pl.BlockSpec((bb, c, hw), lambda i: (i, 0, 0)),
            pl.BlockSpec((n, c), lambda i: (0, 0)),
            pl.BlockSpec((n, c), lambda i: (0, 0)),
        ],
        out_specs=(
            pl.BlockSpec((bb, c, hw), lambda i: (i, 0, 0)),
            pl.BlockSpec((bb, hw, c), lambda i: (i, 0, 0)),
            pl.BlockSpec((bb, hw, c), lambda i: (i, 0, 0)),
            pl.BlockSpec((bb, hw, c), lambda i: (i, 0, 0)),
            pl.BlockSpec((bb, 1, hw), lambda i: (i, 0, 0)),
        ),
        out_shape=out_shapes,
        compiler_params=pltpu.CompilerParams(
            dimension_semantics=("parallel",)),
    )(z3, embedding, emb2)

    return (
        zq.reshape(b, c, h, w),
        loss.reshape(b, h, w, c),
        closs.reshape(b, h, w, c),
        qloss.reshape(b, h, w, c),
        idx.reshape(-1),
    )


# speedup vs baseline: 1.0469x; 1.0469x over previous
import jax
import jax.numpy as jnp
from jax.experimental import pallas as pl
from jax.experimental.pallas import tpu as pltpu


def _vq_body(z_ref, emb_ref, emb2_ref, zq_ref, loss_ref, closs_ref, qloss_ref,
             idx_ref):
    E = emb_ref[...]
    E2 = emb2_ref[...]
    e_sq = jnp.sum(E * E, axis=1, keepdims=True)
    for bb in range(z_ref.shape[0]):
        _vq_batch(z_ref, emb_ref, emb2_ref, zq_ref, loss_ref, closs_ref,
                  qloss_ref, idx_ref, E, E2, e_sq, bb)


def _vq_batch(z_ref, emb_ref, emb2_ref, zq_ref, loss_ref, closs_ref, qloss_ref,
              idx_ref, E, E2, e_sq, bb):
    Z = z_ref[bb]
    hw = Z.shape[1]
    n = E.shape[0]
    z_sq = jnp.sum(Z * Z, axis=0, keepdims=True)
    mm2 = jax.lax.dot_general(
        E2, Z, (((1,), (0,)), ((), ())),
        preferred_element_type=jnp.float32)

    nchains = 4
    ngroups = n // 8
    sub = jax.lax.broadcasted_iota(jnp.int32, (8, hw), 0)
    chains = []
    for k in range(nchains):
        run = None
        rid = None
        for r in range(k, ngroups, nchains):
            dr = (z_sq + e_sq[r * 8:(r + 1) * 8]) - mm2[r * 8:(r + 1) * 8]
            if run is None:
                run, rid = dr, jnp.full((8, hw), r, jnp.int32)
            else:
                lt = dr < run
                rid = jnp.where(lt, r, rid)
                run = jnp.minimum(run, dr)
        chains.append((run, rid * 8 + sub))

    def merge(a, b):
        take_b = (b[0] < a[0]) | ((b[0] == a[0]) & (b[1] < a[1]))
        return (jnp.where(take_b, b[0], a[0]),
                jnp.where(take_b, b[1], a[1]))

    run, code = chains[0]
    for ch in chains[1:]:
        run, code = merge((run, code), ch)

    vmin = jnp.min(run, axis=0, keepdims=True)
    idx = jnp.min(jnp.where(run == vmin, code, jnp.int32(n)),
                  axis=0, keepdims=True)

    iota = jax.lax.broadcasted_iota(jnp.int32, (n, hw), 0)
    onehot = (iota == idx).astype(jnp.float32)
    zq_t = jax.lax.dot_general(
        E, onehot, (((0,), (0,)), ((), ())),
        preferred_element_type=jnp.float32)
    zq_ref[bb] = zq_t
    diff = zq_t - Z
    sq_t = diff * diff
    sq = sq_t.T
    loss_ref[bb] = 1.25 * sq
    closs_ref[bb] = 0.25 * sq
    qloss_ref[bb] = sq
    idx_ref[bb] = idx


def kernel(z, embedding):
    z = z.astype(jnp.float32)
    b, c, h, w = z.shape
    hw = h * w
    n = embedding.shape[0]
    z3 = z.reshape(b, c, hw)
    emb2 = embedding + embedding
    bb = 2

    out_shapes = (
        jax.ShapeDtypeStruct((b, c, hw), jnp.float32),
        jax.ShapeDtypeStruct((b, hw, c), jnp.float32),
        jax.ShapeDtypeStruct((b, hw, c), jnp.float32),
        jax.ShapeDtypeStruct((b, hw, c), jnp.float32),
        jax.ShapeDtypeStruct((b, 1, hw), jnp.int32),
    )
    zq, loss, closs, qloss, idx = pl.pallas_call(
        _vq_body,
        grid=(b // bb,),
        in_specs=[
            pl.BlockSpec((bb, c, hw), lambda i: (i, 0, 0)),
            pl.BlockSpec((n, c), lambda i: (0, 0)),
            pl.BlockSpec((n, c), lambda i: (0, 0)),
        ],
        out_specs=(
            pl.BlockSpec((bb, c, hw), lambda i: (i, 0, 0)),
            pl.BlockSpec((bb, hw, c), lambda i: (i, 0, 0)),
            pl.BlockSpec((bb, hw, c), lambda i: (i, 0, 0)),
            pl.BlockSpec((bb, hw, c), lambda i: (i, 0, 0)),
            pl.BlockSpec((bb, 1, hw), lambda i: (i, 0, 0)),
        ),
        out_shape=out_shapes,
        compiler_params=pltpu.CompilerParams(
            dimension_semantics=("parallel",)),
    )(z3, embedding, emb2)

    return (
        zq.reshape(b, c, h, w),
        loss.reshape(b, h, w, c),
        closs.reshape(b, h, w, c),
        qloss.reshape(b, h, w, c),
        idx.reshape(-1),
    )
